# Initial kernel scaffold; baseline (speedup 1.0000x reference)
import jax
import jax.numpy as jnp
from jax.experimental import pallas as pl


def kernel(x, edge_index, W1, b1, W2, b2):
    raise NotImplementedError("write your pallas kernel here")


# speedup vs baseline: 11.0054x; 11.0054x over previous
import jax
import jax.numpy as jnp
from jax import lax
from jax.experimental import pallas as pl
from jax.experimental.pallas import tpu as pltpu
from jax.experimental.pallas import tpu_sc as plsc

NC = 2
NS = 16
NW = NC * NS
CHUNK = 128
NUM_HYPEREDGES = 10000


def _sc_propagate(table, gidx, sidx, n_out, with_deg):
    E = gidx.shape[0]
    T, F = table.shape
    assert E % NW == 0
    ept = E // NW
    full, rem = divmod(ept, CHUNK)
    assert ept % 8 == 0
    rpt = -(-n_out // NS // 8) * 8
    n_pad = NS * rpt
    dpt = -(-T // NS // 16) * 16 if with_deg else 0
    bpt = -(-n_out // NS // 16) * 16 if with_deg else 0
    t_pad, b_pad = NS * dpt, NS * bpt
    zlen = max(dpt, bpt)

    out_type = [jax.ShapeDtypeStruct((NC, n_pad, F), jnp.float32)]
    if with_deg:
        out_type += [jax.ShapeDtypeStruct((NC * t_pad,), jnp.float32),
                     jax.ShapeDtypeStruct((NC * b_pad,), jnp.float32)]

    scratch = [pltpu.VMEM((CHUNK,), jnp.int32),
               pltpu.VMEM((CHUNK,), jnp.int32),
               pltpu.VMEM((CHUNK, F), jnp.float32),
               pltpu.VMEM_SHARED((n_pad, F), jnp.float32)]
    if rem:
        scratch += [pltpu.VMEM((rem,), jnp.int32),
                    pltpu.VMEM((rem,), jnp.int32)]
    if with_deg:
        scratch += [pltpu.VMEM((CHUNK,), jnp.float32),
                    pltpu.VMEM((zlen,), jnp.float32),
                    pltpu.VMEM_SHARED((t_pad,), jnp.float32),
                    pltpu.VMEM_SHARED((b_pad,), jnp.float32)]

    mesh = plsc.VectorSubcoreMesh(core_axis_name="c", subcore_axis_name="s",
                                  num_cores=NC, num_subcores=NS)

    def body(*refs):
        it = iter(refs)
        table_r, gidx_r, sidx_r, z2_r = (next(it) for _ in range(4))
        p_r = next(it)
        if with_deg:
            dg_r, bg_r = next(it), next(it)
        gv, sv, rows, acc = (next(it) for _ in range(4))
        if rem:
            gv2, sv2 = next(it), next(it)
        if with_deg:
            ones, zbuf, acc_d, acc_b = next(it), next(it), next(it), next(it)

        c = lax.axis_index("c")
        s = lax.axis_index("s")
        wid = c * NS + s

        pltpu.sync_copy(z2_r, acc.at[pl.ds(s * rpt, rpt)])
        if with_deg:
            for i in range(zlen // 16):
                zbuf[pl.ds(i * 16, 16)] = jnp.zeros((16,), jnp.float32)
            pltpu.sync_copy(zbuf.at[pl.ds(0, dpt)], acc_d.at[pl.ds(s * dpt, dpt)])
            pltpu.sync_copy(zbuf.at[pl.ds(0, bpt)], acc_b.at[pl.ds(s * bpt, bpt)])
            for i in range(CHUNK // 16):
                ones[pl.ds(i * 16, 16)] = jnp.ones((16,), jnp.float32)
        plsc.subcore_barrier()

        base = wid * ept

        def step(j, carry):
            off = pl.multiple_of(base + j * CHUNK, 8)
            pltpu.sync_copy(gidx_r.at[pl.ds(off, CHUNK)], gv)
            pltpu.sync_copy(sidx_r.at[pl.ds(off, CHUNK)], sv)
            pltpu.sync_copy(table_r.at[gv], rows)
            pltpu.sync_copy(rows, acc.at[sv], add=True)
            if with_deg:
                pltpu.sync_copy(ones, acc_d.at[gv], add=True)
                pltpu.sync_copy(ones, acc_b.at[sv], add=True)
            return carry

        lax.fori_loop(0, full, step, 0)

        if rem:
            off = pl.multiple_of(base + full * CHUNK, 8)
            pltpu.sync_copy(gidx_r.at[pl.ds(off, rem)], gv2)
            pltpu.sync_copy(sidx_r.at[pl.ds(off, rem)], sv2)
            pltpu.sync_copy(table_r.at[gv2], rows.at[pl.ds(0, rem)])
            pltpu.sync_copy(rows.at[pl.ds(0, rem)], acc.at[sv2], add=True)
            if with_deg:
                pltpu.sync_copy(ones.at[pl.ds(0, rem)], acc_d.at[gv2], add=True)
                pltpu.sync_copy(ones.at[pl.ds(0, rem)], acc_b.at[sv2], add=True)

        plsc.subcore_barrier()
        pltpu.sync_copy(acc.at[pl.ds(s * rpt, rpt)], p_r.at[c, pl.ds(s * rpt, rpt)])
        if with_deg:
            pltpu.sync_copy(acc_d.at[pl.ds(s * dpt, dpt)], zbuf.at[pl.ds(0, dpt)])
            pltpu.sync_copy(zbuf.at[pl.ds(0, dpt)],
                            dg_r.at[pl.ds(pl.multiple_of(c * t_pad + s * dpt, 8), dpt)])
            pltpu.sync_copy(acc_b.at[pl.ds(s * bpt, bpt)], zbuf.at[pl.ds(0, bpt)])
            pltpu.sync_copy(zbuf.at[pl.ds(0, bpt)],
                            bg_r.at[pl.ds(pl.multiple_of(c * b_pad + s * bpt, 8), bpt)])

    args = [table, gidx, sidx, jnp.zeros((rpt, F), jnp.float32)]

    f = pl.kernel(body, out_type=tuple(out_type), mesh=mesh,
                  scratch_types=scratch,
                  compiler_params=pltpu.CompilerParams(use_tc_tiling_on_sc=False))
    out = f(*args)
    return list(out) if isinstance(out, (tuple, list)) else [out]


_BLK = 1000


def _tc_mm(x, w):
    n, f = x.shape
    f2 = w.shape[1]

    def body(x_r, w_r, o_r):
        o_r[...] = jnp.dot(x_r[...], w_r[...], preferred_element_type=jnp.float32)

    return pl.pallas_call(
        body,
        grid=(n // _BLK,),
        in_specs=[pl.BlockSpec((_BLK, f), lambda i: (i, 0)),
                  pl.BlockSpec((f, f2), lambda i: (0, 0))],
        out_specs=pl.BlockSpec((_BLK, f2), lambda i: (i, 0)),
        out_shape=jax.ShapeDtypeStruct((n, f2), jnp.float32),
    )(x, w)


def _tc_combine_scale(p, degp, bias=None):
    f = p.shape[2]
    n = degp.shape[1]

    def body(p_r, d_r, *rest):
        o_r = rest[-1]
        ssum = p_r[0] + p_r[1]
        deg = d_r[0] + d_r[1]
        o = ssum / jnp.maximum(deg, 1.0)
        if bias is not None:
            o = o + rest[0][...]
        o_r[...] = o

    in_specs = [pl.BlockSpec((NC, _BLK, f), lambda i: (0, i, 0)),
                pl.BlockSpec((NC, _BLK, 1), lambda i: (0, i, 0))]
    args = [p, degp]
    if bias is not None:
        in_specs.append(pl.BlockSpec((1, f), lambda i: (0, 0)))
        args.append(bias)
    return pl.pallas_call(
        body,
        grid=(n // _BLK,),
        in_specs=in_specs,
        out_specs=pl.BlockSpec((_BLK, f), lambda i: (i, 0)),
        out_shape=jax.ShapeDtypeStruct((n, f), jnp.float32),
    )(*args)


def _tc_combine_elu_mm(q, degp, b, w):
    f = q.shape[2]
    n = degp.shape[1]
    f2 = w.shape[1]

    def body(q_r, d_r, b_r, w_r, o_r):
        ssum = q_r[0] + q_r[1]
        deg = d_r[0] + d_r[1]
        h = ssum / jnp.maximum(deg, 1.0) + b_r[...]
        h = jnp.where(h > 0, h, jnp.exp(jnp.minimum(h, 0.0)) - 1.0)
        o_r[...] = jnp.dot(h, w_r[...], preferred_element_type=jnp.float32)

    return pl.pallas_call(
        body,
        grid=(n // _BLK,),
        in_specs=[pl.BlockSpec((NC, _BLK, f), lambda i: (0, i, 0)),
                  pl.BlockSpec((NC, _BLK, 1), lambda i: (0, i, 0)),
                  pl.BlockSpec((1, f), lambda i: (0, 0)),
                  pl.BlockSpec((f, f2), lambda i: (0, 0))],
        out_specs=pl.BlockSpec((_BLK, f2), lambda i: (i, 0)),
        out_shape=jax.ShapeDtypeStruct((n, f2), jnp.float32),
    )(q, degp, b, w)


def kernel(x, edge_index, W1, b1, W2, b2):
    n, _ = x.shape
    m = NUM_HYPEREDGES
    row = edge_index[0]
    col = edge_index[1]

    xw = _tc_mm(x, W1)
    p1, dg, bg = _sc_propagate(xw, row, col, m, with_deg=True)
    dg3 = dg.reshape(NC, -1)[:, :n].reshape(NC, n, 1)
    bg3 = bg.reshape(NC, -1)[:, :m].reshape(NC, m, 1)
    out_e = _tc_combine_scale(p1, bg3)
    (q1,) = _sc_propagate(out_e, col, row, n, with_deg=False)
    h2 = _tc_combine_elu_mm(q1, dg3, b1.reshape(1, -1), W2)

    (p2,) = _sc_propagate(h2, row, col, m, with_deg=False)
    out_e2 = _tc_combine_scale(p2, bg3)
    (q2,) = _sc_propagate(out_e2, col, row, n, with_deg=False)
    out = _tc_combine_scale(q2, dg3, bias=b2.reshape(1, -1))
    return out


# speedup vs baseline: 16.6266x; 1.5108x over previous
import jax
import jax.numpy as jnp
from jax import lax
from jax.experimental import pallas as pl
from jax.experimental.pallas import tpu as pltpu
from jax.experimental.pallas import tpu_sc as plsc

NC = 2
NS = 16
NW = NC * NS
CHUNK = 128
NUM_HYPEREDGES = 10000


def _sc_propagate(table, gidx, sidx, n_out, with_deg):
    E = gidx.shape[0]
    T, F = table.shape
    assert E % NW == 0
    ept = E // NW
    full, rem = divmod(ept, CHUNK)
    assert ept % 8 == 0
    rpt = -(-n_out // NS // 8) * 8
    n_pad = NS * rpt
    dpt = -(-T // NS // 16) * 16 if with_deg else 0
    bpt = -(-n_out // NS // 16) * 16 if with_deg else 0
    t_pad, b_pad = NS * dpt, NS * bpt
    zlen = max(dpt, bpt)

    out_type = [jax.ShapeDtypeStruct((NC, n_pad, F), jnp.float32)]
    if with_deg:
        out_type += [jax.ShapeDtypeStruct((NC * t_pad,), jnp.float32),
                     jax.ShapeDtypeStruct((NC * b_pad,), jnp.float32)]

    scratch = [pltpu.VMEM((CHUNK,), jnp.int32),
               pltpu.VMEM((CHUNK,), jnp.int32),
               pltpu.VMEM((CHUNK, F), jnp.float32),
               pltpu.VMEM((CHUNK,), jnp.int32),
               pltpu.VMEM((CHUNK,), jnp.int32),
               pltpu.VMEM((CHUNK, F), jnp.float32),
               pltpu.SemaphoreType.DMA,
               pltpu.SemaphoreType.DMA,
               pltpu.SemaphoreType.DMA,
               pltpu.SemaphoreType.DMA,
               pltpu.VMEM_SHARED((n_pad, F), jnp.float32)]
    if rem:
        scratch += [pltpu.VMEM((rem,), jnp.int32),
                    pltpu.VMEM((rem,), jnp.int32)]
    if with_deg:
        scratch += [pltpu.VMEM((CHUNK,), jnp.float32),
                    pltpu.VMEM((zlen,), jnp.float32),
                    pltpu.VMEM_SHARED((t_pad,), jnp.float32),
                    pltpu.VMEM_SHARED((b_pad,), jnp.float32)]

    mesh = plsc.VectorSubcoreMesh(core_axis_name="c", subcore_axis_name="s",
                                  num_cores=NC, num_subcores=NS)

    def body(*refs):
        it = iter(refs)
        table_r, gidx_r, sidx_r, z2_r = (next(it) for _ in range(4))
        p_r = next(it)
        if with_deg:
            dg_r, bg_r = next(it), next(it)
        (gvA, svA, rowsA, gvB, svB, rowsB,
         gsemA, gsemB, ssemA, ssemB, acc) = (next(it) for _ in range(11))
        if rem:
            gv2, sv2 = next(it), next(it)
        if with_deg:
            ones, zbuf, acc_d, acc_b = next(it), next(it), next(it), next(it)

        c = lax.axis_index("c")
        s = lax.axis_index("s")
        wid = c * NS + s

        pltpu.sync_copy(z2_r, acc.at[pl.ds(s * rpt, rpt)])
        if with_deg:
            for i in range(zlen // 16):
                zbuf[pl.ds(i * 16, 16)] = jnp.zeros((16,), jnp.float32)
            pltpu.sync_copy(zbuf.at[pl.ds(0, dpt)], acc_d.at[pl.ds(s * dpt, dpt)])
            pltpu.sync_copy(zbuf.at[pl.ds(0, bpt)], acc_b.at[pl.ds(s * bpt, bpt)])
            for i in range(CHUNK // 16):
                ones[pl.ds(i * 16, 16)] = jnp.ones((16,), jnp.float32)
        plsc.subcore_barrier()

        base = wid * ept
        bufs = ((gvA, svA, rowsA, gsemA, ssemA),
                (gvB, svB, rowsB, gsemB, ssemB))

        def load_idx(j, gvx, svx):
            off = pl.multiple_of(base + j * CHUNK, 8)
            pltpu.sync_copy(gidx_r.at[pl.ds(off, CHUNK)], gvx)
            pltpu.sync_copy(sidx_r.at[pl.ds(off, CHUNK)], svx)

        def half(j, bx, by):
            gvx, svx, rowsx, gsemx, ssemx = bx
            gvy, svy, rowsy, gsemy, ssemy = by

            @pl.when(j >= 1)
            def _():
                pltpu.make_async_copy(rowsy, acc.at[svy], ssemy).wait()

            @pl.when(j + 1 < full)
            def _():
                load_idx(j + 1, gvy, svy)

            pltpu.make_async_copy(table_r.at[gvx], rowsx, gsemx).wait()

            @pl.when(j + 1 < full)
            def _():
                pltpu.async_copy(table_r.at[gvy], rowsy, gsemy)

            pltpu.async_copy(rowsx, acc.at[svx], ssemx, add=True)
            if with_deg:
                pltpu.sync_copy(ones, acc_d.at[gvx], add=True)
                pltpu.sync_copy(ones, acc_b.at[svx], add=True)

        load_idx(0, gvA, svA)
        pltpu.async_copy(table_r.at[gvA], rowsA, gsemA)

        def step2(k, carry):
            half(2 * k, bufs[0], bufs[1])
            half(2 * k + 1, bufs[1], bufs[0])
            return carry

        lax.fori_loop(0, full // 2, step2, 0)
        if full % 2:
            half(full - 1, bufs[0], bufs[1])

        _, svL, rowsL, _, ssemL = bufs[(full - 1) % 2]
        pltpu.make_async_copy(rowsL, acc.at[svL], ssemL).wait()

        if rem:
            off = pl.multiple_of(base + full * CHUNK, 8)
            pltpu.sync_copy(gidx_r.at[pl.ds(off, rem)], gv2)
            pltpu.sync_copy(sidx_r.at[pl.ds(off, rem)], sv2)
            pltpu.sync_copy(table_r.at[gv2], rowsA.at[pl.ds(0, rem)])
            pltpu.sync_copy(rowsA.at[pl.ds(0, rem)], acc.at[sv2], add=True)
            if with_deg:
                pltpu.sync_copy(ones.at[pl.ds(0, rem)], acc_d.at[gv2], add=True)
                pltpu.sync_copy(ones.at[pl.ds(0, rem)], acc_b.at[sv2], add=True)

        plsc.subcore_barrier()
        pltpu.sync_copy(acc.at[pl.ds(s * rpt, rpt)], p_r.at[c, pl.ds(s * rpt, rpt)])
        if with_deg:
            pltpu.sync_copy(acc_d.at[pl.ds(s * dpt, dpt)], zbuf.at[pl.ds(0, dpt)])
            pltpu.sync_copy(zbuf.at[pl.ds(0, dpt)],
                            dg_r.at[pl.ds(pl.multiple_of(c * t_pad + s * dpt, 8), dpt)])
            pltpu.sync_copy(acc_b.at[pl.ds(s * bpt, bpt)], zbuf.at[pl.ds(0, bpt)])
            pltpu.sync_copy(zbuf.at[pl.ds(0, bpt)],
                            bg_r.at[pl.ds(pl.multiple_of(c * b_pad + s * bpt, 8), bpt)])

    args = [table, gidx, sidx, jnp.zeros((rpt, F), jnp.float32)]

    f = pl.kernel(body, out_type=tuple(out_type), mesh=mesh,
                  scratch_types=scratch,
                  compiler_params=pltpu.CompilerParams(use_tc_tiling_on_sc=False))
    out = f(*args)
    return list(out) if isinstance(out, (tuple, list)) else [out]


_BLK = 1000


def _tc_mm(x, w):
    n, f = x.shape
    f2 = w.shape[1]

    def body(x_r, w_r, o_r):
        o_r[...] = jnp.dot(x_r[...], w_r[...], preferred_element_type=jnp.float32)

    return pl.pallas_call(
        body,
        grid=(n // _BLK,),
        in_specs=[pl.BlockSpec((_BLK, f), lambda i: (i, 0)),
                  pl.BlockSpec((f, f2), lambda i: (0, 0))],
        out_specs=pl.BlockSpec((_BLK, f2), lambda i: (i, 0)),
        out_shape=jax.ShapeDtypeStruct((n, f2), jnp.float32),
    )(x, w)


def _tc_combine_scale(p, degp, bias=None):
    f = p.shape[2]
    n = degp.shape[1]

    def body(p_r, d_r, *rest):
        o_r = rest[-1]
        ssum = p_r[0] + p_r[1]
        deg = d_r[0] + d_r[1]
        o = ssum / jnp.maximum(deg, 1.0)
        if bias is not None:
            o = o + rest[0][...]
        o_r[...] = o

    in_specs = [pl.BlockSpec((NC, _BLK, f), lambda i: (0, i, 0)),
                pl.BlockSpec((NC, _BLK, 1), lambda i: (0, i, 0))]
    args = [p, degp]
    if bias is not None:
        in_specs.append(pl.BlockSpec((1, f), lambda i: (0, 0)))
        args.append(bias)
    return pl.pallas_call(
        body,
        grid=(n // _BLK,),
        in_specs=in_specs,
        out_specs=pl.BlockSpec((_BLK, f), lambda i: (i, 0)),
        out_shape=jax.ShapeDtypeStruct((n, f), jnp.float32),
    )(*args)


def _tc_combine_elu_mm(q, degp, b, w):
    f = q.shape[2]
    n = degp.shape[1]
    f2 = w.shape[1]

    def body(q_r, d_r, b_r, w_r, o_r):
        ssum = q_r[0] + q_r[1]
        deg = d_r[0] + d_r[1]
        h = ssum / jnp.maximum(deg, 1.0) + b_r[...]
        h = jnp.where(h > 0, h, jnp.exp(jnp.minimum(h, 0.0)) - 1.0)
        o_r[...] = jnp.dot(h, w_r[...], preferred_element_type=jnp.float32)

    return pl.pallas_call(
        body,
        grid=(n // _BLK,),
        in_specs=[pl.BlockSpec((NC, _BLK, f), lambda i: (0, i, 0)),
                  pl.BlockSpec((NC, _BLK, 1), lambda i: (0, i, 0)),
                  pl.BlockSpec((1, f), lambda i: (0, 0)),
                  pl.BlockSpec((f, f2), lambda i: (0, 0))],
        out_specs=pl.BlockSpec((_BLK, f2), lambda i: (i, 0)),
        out_shape=jax.ShapeDtypeStruct((n, f2), jnp.float32),
    )(q, degp, b, w)


def kernel(x, edge_index, W1, b1, W2, b2):
    n, _ = x.shape
    m = NUM_HYPEREDGES
    row = edge_index[0]
    col = edge_index[1]

    xw = _tc_mm(x, W1)
    p1, dg, bg = _sc_propagate(xw, row, col, m, with_deg=True)
    dg3 = dg.reshape(NC, -1)[:, :n].reshape(NC, n, 1)
    bg3 = bg.reshape(NC, -1)[:, :m].reshape(NC, m, 1)
    out_e = _tc_combine_scale(p1, bg3)
    (q1,) = _sc_propagate(out_e, col, row, n, with_deg=False)
    h2 = _tc_combine_elu_mm(q1, dg3, b1.reshape(1, -1), W2)

    (p2,) = _sc_propagate(h2, row, col, m, with_deg=False)
    out_e2 = _tc_combine_scale(p2, bg3)
    (q2,) = _sc_propagate(out_e2, col, row, n, with_deg=False)
    out = _tc_combine_scale(q2, dg3, bias=b2.reshape(1, -1))
    return out
